# baseline (speedup 1.0000x reference)
import functools

import jax
import jax.numpy as jnp
from jax import lax
from jax.experimental import pallas as pl
from jax.experimental.pallas import tpu as pltpu
from jax.experimental.pallas import tpu_sc as plsc

NC = 2
NS = 16
NW = NC * NS
W = 8
MBLK = 250


def _sc_mesh():
  return plsc.VectorSubcoreMesh(
      core_axis_name="c", subcore_axis_name="s", num_cores=NC,
      num_subcores=NS)


_SC_PARAMS = pltpu.CompilerParams(use_tc_tiling_on_sc=False,
                                  needs_layout_passes=False)


def _make_sc_fused1(n, e, chunk):
  deg_tile_e = e // NS
  deg_steps = deg_tile_e // chunk
  tile_e = e // NW
  prop_steps = tile_e // chunk
  rpt = n // NS
  nblk = rpt // MBLK
  inner = MBLK * W // 16
  assert deg_steps * chunk == deg_tile_e
  assert prop_steps * chunk == tile_e
  assert nblk * MBLK == rpt

  @functools.partial(
      pl.kernel,
      out_type=(jax.ShapeDtypeStruct((n, W), jnp.float32),
                jax.ShapeDtypeStruct((NC, n, W), jnp.float32)),
      mesh=_sc_mesh(),
      scratch_types=[
          pltpu.VMEM_SHARED((n, W), jnp.float32),
          pltpu.VMEM_SHARED((n, W), jnp.float32),
          pltpu.VMEM((chunk,), jnp.int32),
          pltpu.VMEM((chunk,), jnp.int32),
          pltpu.VMEM((chunk, W), jnp.float32),
          pltpu.VMEM((MBLK, W), jnp.float32),
          pltpu.VMEM((MBLK, W), jnp.float32),
          pltpu.VMEM((MBLK, W), jnp.float32),
          pltpu.SemaphoreType.DMA,
      ],
      compiler_params=_SC_PARAMS,
  )
  def fused_kernel(src_hbm, dst_hbm, x8_hbm, zeros_hbm, ones_hbm,
                   dinv_hbm, aggp_hbm, acc_sh, y_sh, src_v, dst_v,
                   rows_v, degb, xb, yb, sem):
    cid = lax.axis_index("c")
    sid = lax.axis_index("s")
    wid = cid * NS + sid
    r0 = sid * rpt
    pltpu.sync_copy(zeros_hbm.at[pl.ds(r0, rpt)], acc_sh.at[pl.ds(r0, rpt)])
    pltpu.sync_copy(ones_hbm, rows_v)
    plsc.subcore_barrier()

    def dstep(j, c):
      base = sid * deg_tile_e + j * chunk
      pltpu.sync_copy(dst_hbm.at[pl.ds(base, chunk)], dst_v)
      pltpu.sync_copy(rows_v, acc_sh.at[dst_v], add=True)
      return c

    lax.fori_loop(0, deg_steps, dstep, 0)
    plsc.subcore_barrier()

    iota = lax.iota(jnp.int32, 16)
    rbase = jnp.right_shift(iota, 3)
    col = jnp.bitwise_and(iota, 7)

    def math_blk(b, c):
      rb = r0 + b * MBLK
      pltpu.sync_copy(acc_sh.at[pl.ds(rb, MBLK)], degb)
      pltpu.sync_copy(x8_hbm.at[pl.ds(rb, MBLK)], xb)

      def mstep(k, c2):
        row = rbase + 2 * k
        dv = plsc.load_gather(degb, [row, col]) + 1.0
        xv = plsc.load_gather(xb, [row, col])
        iv = jnp.int32(0x5F3759DF) - jnp.right_shift(
            plsc.bitcast(dv, jnp.int32), 1)
        g = plsc.bitcast(iv, jnp.float32)
        g = g * (1.5 - 0.5 * dv * g * g)
        g = g * (1.5 - 0.5 * dv * g * g)
        g = g * (1.5 - 0.5 * dv * g * g)
        plsc.store_scatter(degb, [row, col], g)
        plsc.store_scatter(yb, [row, col], g * xv)
        return c2

      lax.fori_loop(0, inner, mstep, 0)
      pltpu.sync_copy(yb, y_sh.at[pl.ds(rb, MBLK)])

      @pl.when(cid == 0)
      def _():
        pltpu.sync_copy(degb, dinv_hbm.at[pl.ds(rb, MBLK)])
      return c

    lax.fori_loop(0, nblk, math_blk, 0)
    pltpu.sync_copy(zeros_hbm.at[pl.ds(r0, rpt)], acc_sh.at[pl.ds(r0, rpt)])
    plsc.subcore_barrier()

    def pstep(j, c):
      base = wid * tile_e + j * chunk
      pltpu.sync_copy(src_hbm.at[pl.ds(base, chunk)], src_v)
      pltpu.sync_copy(dst_hbm.at[pl.ds(base, chunk)], dst_v)
      pltpu.async_copy(y_sh.at[src_v], rows_v, sem).wait()
      pltpu.sync_copy(rows_v, acc_sh.at[dst_v], add=True)
      return c

    lax.fori_loop(0, prop_steps, pstep, 0)
    plsc.subcore_barrier()
    pltpu.sync_copy(acc_sh.at[pl.ds(r0, rpt)],
                    aggp_hbm.at[cid, pl.ds(r0, rpt)])

  return fused_kernel


def _make_sc_prop(n, e, chunk):
  tile_e = e // NW
  nsteps = tile_e // chunk
  assert nsteps * chunk == tile_e
  rpt = n // NS

  @functools.partial(
      pl.kernel,
      out_type=jax.ShapeDtypeStruct((NC, n, W), jnp.float32),
      mesh=_sc_mesh(),
      scratch_types=[
          pltpu.VMEM_SHARED((n, W), jnp.float32),
          pltpu.VMEM_SHARED((n, W), jnp.float32),
          pltpu.VMEM((chunk,), jnp.int32),
          pltpu.VMEM((chunk,), jnp.int32),
          pltpu.VMEM((chunk, W), jnp.float32),
          pltpu.SemaphoreType.DMA,
      ],
      compiler_params=_SC_PARAMS,
  )
  def prop_kernel(src_hbm, dst_hbm, y_hbm, zeros_hbm, out_hbm, acc_sh,
                  y_sh, src_v, dst_v, rows_v, sem):
    cid = lax.axis_index("c")
    sid = lax.axis_index("s")
    wid = cid * NS + sid
    r0 = sid * rpt
    pltpu.sync_copy(zeros_hbm.at[pl.ds(r0, rpt)], acc_sh.at[pl.ds(r0, rpt)])
    pltpu.sync_copy(y_hbm.at[pl.ds(r0, rpt)], y_sh.at[pl.ds(r0, rpt)])
    plsc.subcore_barrier()

    def pstep(j, c):
      base = wid * tile_e + j * chunk
      pltpu.sync_copy(src_hbm.at[pl.ds(base, chunk)], src_v)
      pltpu.sync_copy(dst_hbm.at[pl.ds(base, chunk)], dst_v)
      pltpu.async_copy(y_sh.at[src_v], rows_v, sem).wait()
      pltpu.sync_copy(rows_v, acc_sh.at[dst_v], add=True)
      return c

    lax.fori_loop(0, nsteps, pstep, 0)
    plsc.subcore_barrier()
    pltpu.sync_copy(acc_sh.at[pl.ds(r0, rpt)],
                    out_hbm.at[cid, pl.ds(r0, rpt)])

  return prop_kernel


def _tc_mid(aggp, x8, dinv8, W1, b1, W2p, bn):
  n = x8.shape[0]
  nblk = n // bn

  def body(aggp_ref, x8_ref, dinv8_ref, w1_ref, b1_ref, w2_ref, y2_ref):
    dinv8 = dinv8_ref[...]
    z = dinv8 * (aggp_ref[0] + aggp_ref[1] + dinv8 * x8_ref[...])
    w1 = w1_ref[...]
    h = (z[:, 0:1] * w1[0:1, :] + z[:, 1:2] * w1[1:2, :]
         + z[:, 2:3] * w1[2:3, :] + z[:, 3:4] * w1[3:4, :])
    h = jnp.maximum(h + b1_ref[...], 0.0)
    y2_ref[...] = dinv8 * jnp.dot(h, w2_ref[...],
                                  preferred_element_type=jnp.float32)

  return pl.pallas_call(
      body,
      grid=(nblk,),
      in_specs=[
          pl.BlockSpec((2, bn, W), lambda i: (0, i, 0)),
          pl.BlockSpec((bn, W), lambda i: (i, 0)),
          pl.BlockSpec((bn, W), lambda i: (i, 0)),
          pl.BlockSpec((4, 128), lambda i: (0, 0)),
          pl.BlockSpec((1, 128), lambda i: (0, 0)),
          pl.BlockSpec((128, W), lambda i: (0, 0)),
      ],
      out_specs=pl.BlockSpec((bn, W), lambda i: (i, 0)),
      out_shape=jax.ShapeDtypeStruct((n, W), jnp.float32),
  )(aggp, x8, dinv8, W1, b1, W2p)


def _tc_final(aggp_flat, y2_flat, dinv8_flat, b2row):
  def body(aggp_ref, y2_ref, dinv8_ref, b2_ref, out_ref):
    out_ref[...] = (dinv8_ref[...] * (aggp_ref[0] + aggp_ref[1]
                                      + y2_ref[...]) + b2_ref[...])

  m = y2_flat.shape[0]
  return pl.pallas_call(
      body,
      out_shape=jax.ShapeDtypeStruct((m, 128), jnp.float32),
  )(aggp_flat, y2_flat, dinv8_flat, b2row)


@jax.jit
def kernel(x, edge_index, W1, b1, W2, b2):
  n, in_dim = x.shape
  e = edge_index.shape[1]
  assert in_dim == 4
  m = n * W // 128
  chunk = 2000
  src = edge_index[0]
  dst = edge_index[1]
  x8 = jnp.pad(x, ((0, 0), (0, W - in_dim)))
  W2p = jnp.pad(W2, ((0, 0), (0, W - in_dim)))
  zeros8 = jnp.zeros((n, W), jnp.float32)
  ones8 = jnp.ones((chunk, W), jnp.float32)
  b2row = jnp.reshape(jnp.tile(jnp.pad(b2, (0, W - in_dim)), 128 // W),
                      (1, 128))

  dinv8, agg1p = _make_sc_fused1(n, e, chunk)(src, dst, x8, zeros8, ones8)
  y2 = _tc_mid(agg1p, x8, dinv8, W1, b1.reshape(1, 128), W2p, 2000)
  agg2p = _make_sc_prop(n, e, chunk)(src, dst, y2, zeros8)
  out_f = _tc_final(agg2p.reshape(2, m, 128), y2.reshape(m, 128),
                    dinv8.reshape(m, 128), b2row)
  return out_f.reshape(n, W)[:, :in_dim]


# speedup vs baseline: 1.2287x; 1.2287x over previous
import functools

import jax
import jax.numpy as jnp
from jax import lax
from jax.experimental import pallas as pl
from jax.experimental.pallas import tpu as pltpu
from jax.experimental.pallas import tpu_sc as plsc

NC = 2
NS = 16
NW = NC * NS
W = 8
MBLK = 250


def _sc_mesh():
  return plsc.VectorSubcoreMesh(
      core_axis_name="c", subcore_axis_name="s", num_cores=NC,
      num_subcores=NS)


_SC_PARAMS = pltpu.CompilerParams(use_tc_tiling_on_sc=False,
                                  needs_layout_passes=False)


def _make_sc_fused1(n, e, chunk):
  deg_tile_e = e // NS
  deg_steps = deg_tile_e // chunk
  tile_e = e // NW
  prop_steps = tile_e // chunk
  rpt = n // NS
  nblk = rpt // MBLK
  inner = MBLK * W // 16
  assert deg_steps * chunk == deg_tile_e
  assert prop_steps * chunk == tile_e
  assert nblk * MBLK == rpt

  @functools.partial(
      pl.kernel,
      out_type=(jax.ShapeDtypeStruct((n, W), jnp.float32),
                jax.ShapeDtypeStruct((n, W), jnp.float32),
                jax.ShapeDtypeStruct((NC, n, W), jnp.float32)),
      mesh=_sc_mesh(),
      scratch_types=[
          pltpu.VMEM_SHARED((n, W), jnp.float32),
          pltpu.VMEM_SHARED((n, W), jnp.float32),
          pltpu.VMEM((chunk,), jnp.int32),
          pltpu.VMEM((chunk,), jnp.int32),
          pltpu.VMEM((chunk, W), jnp.float32),
          pltpu.VMEM((MBLK, W), jnp.float32),
          pltpu.VMEM((MBLK, W), jnp.float32),
          pltpu.VMEM((MBLK, W), jnp.float32),
          pltpu.SemaphoreType.DMA,
      ],
      compiler_params=_SC_PARAMS,
  )
  def fused_kernel(src_hbm, dst_hbm, x8_hbm, zeros_hbm, ones_hbm,
                   dinv_hbm, y1_hbm, aggp_hbm, acc_sh, y_sh, src_v,
                   dst_v, rows_v, degb, xb, yb, sem):
    cid = lax.axis_index("c")
    sid = lax.axis_index("s")
    wid = cid * NS + sid
    r0 = sid * rpt
    pltpu.sync_copy(zeros_hbm.at[pl.ds(r0, rpt)], acc_sh.at[pl.ds(r0, rpt)])
    pltpu.sync_copy(ones_hbm, rows_v)
    plsc.subcore_barrier()

    def dstep(j, c):
      base = sid * deg_tile_e + j * chunk
      pltpu.sync_copy(dst_hbm.at[pl.ds(base, chunk)], dst_v)
      pltpu.sync_copy(rows_v, acc_sh.at[dst_v], add=True)
      return c

    lax.fori_loop(0, deg_steps, dstep, 0)
    plsc.subcore_barrier()

    iota = lax.iota(jnp.int32, 16)
    rbase = jnp.right_shift(iota, 3)
    col = jnp.bitwise_and(iota, 7)

    def math_blk(b, c):
      rb = r0 + b * MBLK
      pltpu.sync_copy(acc_sh.at[pl.ds(rb, MBLK)], degb)
      pltpu.sync_copy(x8_hbm.at[pl.ds(rb, MBLK)], xb)

      def mstep(k, c2):
        row = rbase + 2 * k
        dv = plsc.load_gather(degb, [row, col]) + 1.0
        xv = plsc.load_gather(xb, [row, col])
        iv = jnp.int32(0x5F3759DF) - jnp.right_shift(
            plsc.bitcast(dv, jnp.int32), 1)
        g = plsc.bitcast(iv, jnp.float32)
        g = g * (1.5 - 0.5 * dv * g * g)
        g = g * (1.5 - 0.5 * dv * g * g)
        g = g * (1.5 - 0.5 * dv * g * g)
        plsc.store_scatter(degb, [row, col], g)
        plsc.store_scatter(yb, [row, col], g * xv)
        return c2

      lax.fori_loop(0, inner, mstep, 0)
      pltpu.sync_copy(yb, y_sh.at[pl.ds(rb, MBLK)])

      @pl.when(cid == 0)
      def _():
        pltpu.sync_copy(degb, dinv_hbm.at[pl.ds(rb, MBLK)])
        pltpu.sync_copy(yb, y1_hbm.at[pl.ds(rb, MBLK)])
      return c

    lax.fori_loop(0, nblk, math_blk, 0)
    pltpu.sync_copy(zeros_hbm.at[pl.ds(r0, rpt)], acc_sh.at[pl.ds(r0, rpt)])
    plsc.subcore_barrier()

    def pstep(j, c):
      base = wid * tile_e + j * chunk
      pltpu.sync_copy(src_hbm.at[pl.ds(base, chunk)], src_v)
      pltpu.sync_copy(dst_hbm.at[pl.ds(base, chunk)], dst_v)
      pltpu.async_copy(y_sh.at[src_v], rows_v, sem).wait()
      pltpu.sync_copy(rows_v, acc_sh.at[dst_v], add=True)
      return c

    lax.fori_loop(0, prop_steps, pstep, 0)
    plsc.subcore_barrier()
    pltpu.sync_copy(acc_sh.at[pl.ds(r0, rpt)],
                    aggp_hbm.at[cid, pl.ds(r0, rpt)])

  return fused_kernel


def _make_sc_prop(n, e, chunk):
  tile_e = e // NW
  nsteps = tile_e // chunk
  assert nsteps * chunk == tile_e
  rpt = n // NS

  @functools.partial(
      pl.kernel,
      out_type=jax.ShapeDtypeStruct((NC, n, W), jnp.float32),
      mesh=_sc_mesh(),
      scratch_types=[
          pltpu.VMEM_SHARED((n, W), jnp.float32),
          pltpu.VMEM_SHARED((n, W), jnp.float32),
          pltpu.VMEM((chunk,), jnp.int32),
          pltpu.VMEM((chunk,), jnp.int32),
          pltpu.VMEM((chunk, W), jnp.float32),
          pltpu.SemaphoreType.DMA,
      ],
      compiler_params=_SC_PARAMS,
  )
  def prop_kernel(src_hbm, dst_hbm, y_hbm, zeros_hbm, out_hbm, acc_sh,
                  y_sh, src_v, dst_v, rows_v, sem):
    cid = lax.axis_index("c")
    sid = lax.axis_index("s")
    wid = cid * NS + sid
    r0 = sid * rpt
    pltpu.sync_copy(zeros_hbm.at[pl.ds(r0, rpt)], acc_sh.at[pl.ds(r0, rpt)])
    pltpu.sync_copy(y_hbm.at[pl.ds(r0, rpt)], y_sh.at[pl.ds(r0, rpt)])
    plsc.subcore_barrier()

    def pstep(j, c):
      base = wid * tile_e + j * chunk
      pltpu.sync_copy(src_hbm.at[pl.ds(base, chunk)], src_v)
      pltpu.sync_copy(dst_hbm.at[pl.ds(base, chunk)], dst_v)
      pltpu.async_copy(y_sh.at[src_v], rows_v, sem).wait()
      pltpu.sync_copy(rows_v, acc_sh.at[dst_v], add=True)
      return c

    lax.fori_loop(0, nsteps, pstep, 0)
    plsc.subcore_barrier()
    pltpu.sync_copy(acc_sh.at[pl.ds(r0, rpt)],
                    out_hbm.at[cid, pl.ds(r0, rpt)])

  return prop_kernel


def _tc_mid(aggp_f, y1_f, dinv_f, W1, b1, W2p):
  m = y1_f.shape[0]

  def body(aggp_ref, y1_ref, dinv_ref, w1_ref, b1_ref, w2_ref, y2_ref):
    dinv = dinv_ref[...]
    z = dinv * (aggp_ref[0] + aggp_ref[1] + y1_ref[...])
    w1 = w1_ref[...]
    b1v = b1_ref[...]
    w2 = w2_ref[...]
    for g in range(16):
      c = 8 * g
      h = (z[:, c:c + 1] * w1[0:1, :] + z[:, c + 1:c + 2] * w1[1:2, :]
           + z[:, c + 2:c + 3] * w1[2:3, :]
           + z[:, c + 3:c + 4] * w1[3:4, :])
      h = jnp.maximum(h + b1v, 0.0)
      y2_ref[:, c:c + W] = dinv[:, c:c + W] * jnp.dot(
          h, w2, preferred_element_type=jnp.float32)

  return pl.pallas_call(
      body,
      out_shape=jax.ShapeDtypeStruct((m, 128), jnp.float32),
  )(aggp_f, y1_f, dinv_f, W1, b1, W2p)


def _tc_final(aggp_f, y2_f, dinv_f, b2row):
  def body(aggp_ref, y2_ref, dinv_ref, b2_ref, out_ref):
    out_ref[...] = (dinv_ref[...] * (aggp_ref[0] + aggp_ref[1]
                                     + y2_ref[...]) + b2_ref[...])

  m = y2_f.shape[0]
  return pl.pallas_call(
      body,
      out_shape=jax.ShapeDtypeStruct((m, 128), jnp.float32),
  )(aggp_f, y2_f, dinv_f, b2row)


@jax.jit
def kernel(x, edge_index, W1, b1, W2, b2):
  n, in_dim = x.shape
  e = edge_index.shape[1]
  assert in_dim == 4
  m = n * W // 128
  chunk = 2000
  src = edge_index[0]
  dst = edge_index[1]
  x8 = jnp.pad(x, ((0, 0), (0, W - in_dim)))
  W2p = jnp.pad(W2, ((0, 0), (0, W - in_dim)))
  zeros8 = jnp.zeros((n, W), jnp.float32)
  ones8 = jnp.ones((chunk, W), jnp.float32)
  b2row = jnp.reshape(jnp.tile(jnp.pad(b2, (0, W - in_dim)), 128 // W),
                      (1, 128))

  dinv8, y1, agg1p = _make_sc_fused1(n, e, chunk)(src, dst, x8, zeros8,
                                                  ones8)
  y2_f = _tc_mid(agg1p.reshape(2, m, 128), y1.reshape(m, 128),
                 dinv8.reshape(m, 128), W1, b1.reshape(1, 128), W2p)
  agg2p = _make_sc_prop(n, e, chunk)(src, dst, y2_f.reshape(n, W), zeros8)
  out_f = _tc_final(agg2p.reshape(2, m, 128), y2_f,
                    dinv8.reshape(m, 128), b2row)
  return out_f.reshape(n, W)[:, :in_dim]


# speedup vs baseline: 1.5692x; 1.2771x over previous
import functools

import jax
import jax.numpy as jnp
from jax import lax
from jax.experimental import pallas as pl
from jax.experimental.pallas import tpu as pltpu
from jax.experimental.pallas import tpu_sc as plsc

NC = 2
NS = 16
NW = NC * NS
W = 8
MBLK = 250


def _sc_mesh():
  return plsc.VectorSubcoreMesh(
      core_axis_name="c", subcore_axis_name="s", num_cores=NC,
      num_subcores=NS)


_SC_PARAMS = pltpu.CompilerParams(use_tc_tiling_on_sc=False,
                                  needs_layout_passes=False)


def _make_sc_deg(n, e, chunk):
  tile_e = e // NW
  nsteps = tile_e // chunk
  assert nsteps * chunk == tile_e
  rpt = n // NS

  @functools.partial(
      pl.kernel,
      out_type=jax.ShapeDtypeStruct((NC, n, W), jnp.float32),
      mesh=_sc_mesh(),
      scratch_types=[
          pltpu.VMEM_SHARED((n, W), jnp.float32),
          pltpu.VMEM((chunk,), jnp.int32),
          pltpu.VMEM((chunk, W), jnp.float32),
      ],
      compiler_params=_SC_PARAMS,
  )
  def deg_kernel(dst_hbm, zeros_hbm, ones_hbm, out_hbm, acc_sh, idx_v,
                 ones_v):
    cid = lax.axis_index("c")
    sid = lax.axis_index("s")
    wid = cid * NS + sid
    r0 = sid * rpt
    pltpu.sync_copy(zeros_hbm.at[pl.ds(r0, rpt)], acc_sh.at[pl.ds(r0, rpt)])
    pltpu.sync_copy(ones_hbm, ones_v)
    plsc.subcore_barrier()

    def step(j, c):
      base = wid * tile_e + j * chunk
      pltpu.sync_copy(dst_hbm.at[pl.ds(base, chunk)], idx_v)
      pltpu.sync_copy(ones_v, acc_sh.at[idx_v], add=True)
      return c

    lax.fori_loop(0, nsteps, step, 0)
    plsc.subcore_barrier()
    pltpu.sync_copy(acc_sh.at[pl.ds(r0, rpt)],
                    out_hbm.at[cid, pl.ds(r0, rpt)])

  return deg_kernel


def _tc_prep(degp_f, x8_f):
  def body(degp_ref, x_ref, dinv_ref, y1_ref):
    dinv = lax.rsqrt(degp_ref[0] + degp_ref[1] + 1.0)
    dinv_ref[...] = dinv
    y1_ref[...] = dinv * x_ref[...]

  m = x8_f.shape[0]
  return pl.pallas_call(
      body,
      out_shape=(jax.ShapeDtypeStruct((m, 128), jnp.float32),
                 jax.ShapeDtypeStruct((m, 128), jnp.float32)),
  )(degp_f, x8_f)


def _make_sc_prop(n, e, chunk):
  tile_e = e // NW
  nsteps = tile_e // chunk
  assert nsteps * chunk == tile_e
  rpt = n // NS

  @functools.partial(
      pl.kernel,
      out_type=jax.ShapeDtypeStruct((NC, n, W), jnp.float32),
      mesh=_sc_mesh(),
      scratch_types=[
          pltpu.VMEM_SHARED((n, W), jnp.float32),
          pltpu.VMEM_SHARED((n, W), jnp.float32),
          pltpu.VMEM((chunk,), jnp.int32),
          pltpu.VMEM((chunk,), jnp.int32),
          pltpu.VMEM((chunk, W), jnp.float32),
          pltpu.SemaphoreType.DMA,
      ],
      compiler_params=_SC_PARAMS,
  )
  def prop_kernel(src_hbm, dst_hbm, y_hbm, zeros_hbm, out_hbm, acc_sh,
                  y_sh, src_v, dst_v, rows_v, sem):
    cid = lax.axis_index("c")
    sid = lax.axis_index("s")
    wid = cid * NS + sid
    r0 = sid * rpt
    pltpu.sync_copy(zeros_hbm.at[pl.ds(r0, rpt)], acc_sh.at[pl.ds(r0, rpt)])
    pltpu.sync_copy(y_hbm.at[pl.ds(r0, rpt)], y_sh.at[pl.ds(r0, rpt)])
    plsc.subcore_barrier()

    def pstep(j, c):
      base = wid * tile_e + j * chunk
      pltpu.sync_copy(src_hbm.at[pl.ds(base, chunk)], src_v)
      pltpu.sync_copy(dst_hbm.at[pl.ds(base, chunk)], dst_v)
      pltpu.async_copy(y_sh.at[src_v], rows_v, sem).wait()
      pltpu.sync_copy(rows_v, acc_sh.at[dst_v], add=True)
      return c

    lax.fori_loop(0, nsteps, pstep, 0)
    plsc.subcore_barrier()
    pltpu.sync_copy(acc_sh.at[pl.ds(r0, rpt)],
                    out_hbm.at[cid, pl.ds(r0, rpt)])

  return prop_kernel


def _tc_mid(aggp_f, y1_f, dinv_f, W1, b1, W2p):
  m = y1_f.shape[0]

  def body(aggp_ref, y1_ref, dinv_ref, w1_ref, b1_ref, w2_ref, y2_ref):
    dinv = dinv_ref[...]
    z = dinv * (aggp_ref[0] + aggp_ref[1] + y1_ref[...])
    w1 = w1_ref[...]
    b1v = b1_ref[...]
    w2 = w2_ref[...]
    for g in range(16):
      c = 8 * g
      h = (z[:, c:c + 1] * w1[0:1, :] + z[:, c + 1:c + 2] * w1[1:2, :]
           + z[:, c + 2:c + 3] * w1[2:3, :]
           + z[:, c + 3:c + 4] * w1[3:4, :])
      h = jnp.maximum(h + b1v, 0.0)
      y2_ref[:, c:c + W] = dinv[:, c:c + W] * jnp.dot(
          h, w2, preferred_element_type=jnp.float32)

  return pl.pallas_call(
      body,
      out_shape=jax.ShapeDtypeStruct((m, 128), jnp.float32),
  )(aggp_f, y1_f, dinv_f, W1, b1, W2p)


def _tc_final(aggp_f, y2_f, dinv_f, b2row):
  def body(aggp_ref, y2_ref, dinv_ref, b2_ref, out_ref):
    out_ref[...] = (dinv_ref[...] * (aggp_ref[0] + aggp_ref[1]
                                     + y2_ref[...]) + b2_ref[...])

  m = y2_f.shape[0]
  return pl.pallas_call(
      body,
      out_shape=jax.ShapeDtypeStruct((m, 128), jnp.float32),
  )(aggp_f, y2_f, dinv_f, b2row)


@jax.jit
def kernel(x, edge_index, W1, b1, W2, b2):
  n, in_dim = x.shape
  e = edge_index.shape[1]
  assert in_dim == 4
  m = n * W // 128
  chunk = 2000
  src = edge_index[0]
  dst = edge_index[1]
  x8 = jnp.pad(x, ((0, 0), (0, W - in_dim)))
  W2p = jnp.pad(W2, ((0, 0), (0, W - in_dim)))
  zeros8 = jnp.zeros((n, W), jnp.float32)
  ones8 = jnp.ones((chunk, W), jnp.float32)
  b2row = jnp.reshape(jnp.tile(jnp.pad(b2, (0, W - in_dim)), 128 // W),
                      (1, 128))

  prop = _make_sc_prop(n, e, chunk)
  degp = _make_sc_deg(n, e, chunk)(dst, zeros8, ones8)
  dinv_f, y1_f = _tc_prep(degp.reshape(2, m, 128), x8.reshape(m, 128))
  agg1p = prop(src, dst, y1_f.reshape(n, W), zeros8)
  y2_f = _tc_mid(agg1p.reshape(2, m, 128), y1_f, dinv_f, W1,
                 b1.reshape(1, 128), W2p)
  agg2p = prop(src, dst, y2_f.reshape(n, W), zeros8)
  out_f = _tc_final(agg2p.reshape(2, m, 128), y2_f, dinv_f, b2row)
  return out_f.reshape(n, W)[:, :in_dim]
